# Initial kernel scaffold; baseline (speedup 1.0000x reference)
import jax
import jax.numpy as jnp
from jax.experimental import pallas as pl


def kernel(x, edge_index, W1, b1, alpha1, W2, b2, alpha2):
    raise NotImplementedError("write your pallas kernel here")


# speedup vs baseline: 9.1506x; 9.1506x over previous
import functools

import jax
import jax.numpy as jnp
from jax import lax
from jax.experimental import pallas as pl
from jax.experimental.pallas import tpu as pltpu
from jax.experimental.pallas import tpu_sc as plsc

N = 10000
C = 128
E = 320000
NC = 2
NS = 16
NW = NC * NS
B = 128
K = 80
EPT = K * B
E_PAD = NW * EPT
N_PAD = 10112
RPT = N_PAD // NS

_MESH = dict(
    mesh=plsc.VectorSubcoreMesh(
        core_axis_name="c", subcore_axis_name="s", num_cores=NC, num_subcores=NS
    )
)


@functools.partial(
    pl.kernel,
    out_type=jax.ShapeDtypeStruct((NC, N_PAD, C), jnp.float32),
    scratch_types=[
        pltpu.VMEM((K, B), jnp.int32),
        pltpu.VMEM((B, C), jnp.float32),
        pltpu.VMEM_SHARED((N_PAD, C), jnp.float32),
    ],
    **_MESH,
)
def _deg(dst_hbm, zeros_hbm, out_hbm, idx_ref, ones_ref, acc_ref):
    cid = lax.axis_index("c")
    sid = lax.axis_index("s")
    wid = sid * NC + cid
    rows = pl.ds(sid * RPT, RPT)
    pltpu.sync_copy(zeros_hbm.at[rows], acc_ref.at[rows])
    pltpu.sync_copy(dst_hbm.at[wid], idx_ref)
    one = jnp.full((16,), 1.0, jnp.float32)
    for r in range(B):
        for cc in range(C // 16):
            ones_ref[r, pl.ds(cc * 16, 16)] = one
    plsc.subcore_barrier()
    for g in range(K):
        pltpu.sync_copy(ones_ref, acc_ref.at[idx_ref.at[g]], add=True)
    plsc.subcore_barrier()
    pltpu.sync_copy(acc_ref.at[rows], out_hbm.at[cid, rows])


@functools.partial(
    pl.kernel,
    out_type=jax.ShapeDtypeStruct((NC, N_PAD, C), jnp.float32),
    scratch_types=[
        pltpu.VMEM((16, B), jnp.int32),
        pltpu.VMEM((K, B), jnp.int32),
        pltpu.VMEM((2, B, C), jnp.float32),
        pltpu.VMEM_SHARED((N_PAD, C), jnp.float32),
        pltpu.SemaphoreType.DMA,
        pltpu.SemaphoreType.DMA,
    ],
    **_MESH,
)
def _agg(hs_hbm, src_hbm, dst_hbm, zeros_hbm, out_hbm,
         src_ref, dst_ref, rows_ref, acc_ref, sem0, sem1):
    cid = lax.axis_index("c")
    sid = lax.axis_index("s")
    wid = sid * NC + cid
    rows = pl.ds(sid * RPT, RPT)
    pltpu.sync_copy(zeros_hbm.at[rows], acc_ref.at[rows])
    pltpu.sync_copy(dst_hbm.at[wid], dst_ref)
    plsc.subcore_barrier()
    sems = (sem0, sem1)
    for t in range(K // 16):
        pltpu.sync_copy(src_hbm.at[wid, pl.ds(t * 16, 16)], src_ref)
        pltpu.async_copy(hs_hbm.at[src_ref.at[0]], rows_ref.at[0], sems[0])
        for g in range(16):
            nxt = (g + 1) % 2
            if g + 1 < 16:
                pltpu.async_copy(
                    hs_hbm.at[src_ref.at[g + 1]], rows_ref.at[nxt], sems[nxt]
                )
            pltpu.make_async_copy(
                hs_hbm.at[src_ref.at[g]], rows_ref.at[g % 2], sems[g % 2]
            ).wait()
            pltpu.sync_copy(
                rows_ref.at[g % 2], acc_ref.at[dst_ref.at[t * 16 + g]], add=True
            )
    plsc.subcore_barrier()
    pltpu.sync_copy(acc_ref.at[rows], out_hbm.at[cid, rows])


_RB = 1000
_GRID = N // _RB


def _dis_of(degp_ref):
    d = degp_ref[0, :, 0] + degp_ref[1, :, 0] + 1.0
    return lax.rsqrt(d)[:, None]


def _k1_body(x_ref, w_ref, degp_ref, o_ref):
    h = jnp.dot(x_ref[...], w_ref[...], preferred_element_type=jnp.float32)
    o_ref[...] = h * _dis_of(degp_ref)


def _k2_body(p_ref, hs_ref, degp_ref, b_ref, a_ref, w_ref, o_ref):
    dis = _dis_of(degp_ref)
    y = (p_ref[0] + p_ref[1] + hs_ref[...]) * dis + b_ref[...]
    h = jnp.where(y >= 0.0, y, a_ref[...] * y)
    o_ref[...] = jnp.dot(h, w_ref[...], preferred_element_type=jnp.float32) * dis


def _k3_body(p_ref, hs_ref, degp_ref, b_ref, a_ref, o_ref):
    y = (p_ref[0] + p_ref[1] + hs_ref[...]) * _dis_of(degp_ref) + b_ref[...]
    o_ref[...] = jnp.where(y >= 0.0, y, a_ref[...] * y)


_row_spec = pl.BlockSpec((_RB, C), lambda i: (i, 0))
_w_spec = pl.BlockSpec((C, C), lambda i: (0, 0))
_vec_spec = pl.BlockSpec((1, C), lambda i: (0, 0))
_degp_spec = pl.BlockSpec((NC, _RB, C), lambda i: (0, i, 0))
_p_spec = pl.BlockSpec((NC, _RB, C), lambda i: (0, i, 0))
_out_shape = jax.ShapeDtypeStruct((N, C), jnp.float32)

_k1 = pl.pallas_call(
    _k1_body,
    grid=(_GRID,),
    in_specs=[_row_spec, _w_spec, _degp_spec],
    out_specs=_row_spec,
    out_shape=_out_shape,
)

_k2 = pl.pallas_call(
    _k2_body,
    grid=(_GRID,),
    in_specs=[_p_spec, _row_spec, _degp_spec, _vec_spec, _vec_spec, _w_spec],
    out_specs=_row_spec,
    out_shape=_out_shape,
)

_k3 = pl.pallas_call(
    _k3_body,
    grid=(_GRID,),
    in_specs=[_p_spec, _row_spec, _degp_spec, _vec_spec, _vec_spec],
    out_specs=_row_spec,
    out_shape=_out_shape,
)


@jax.jit
def kernel(x, edge_index, W1, b1, alpha1, W2, b2, alpha2):
    src = edge_index[0].astype(jnp.int32)
    dst = edge_index[1].astype(jnp.int32)
    npad = E_PAD - E
    src = jnp.concatenate([src, jnp.zeros((npad,), jnp.int32)])
    dst = jnp.concatenate([dst, jnp.full((npad,), N, jnp.int32)])
    src_r = src.reshape(NW, K, B)
    dst_r = dst.reshape(NW, K, B)

    zeros_acc = jnp.zeros((N_PAD, C), jnp.float32)
    b1r = b1.reshape(1, C)
    b2r = b2.reshape(1, C)
    a1r = alpha1.reshape(1, C)
    a2r = alpha2.reshape(1, C)

    degp = _deg(dst_r, zeros_acc)
    hs1 = _k1(x, W1, degp)
    p1 = _agg(hs1, src_r, dst_r, zeros_acc)
    hs2 = _k2(p1, hs1, degp, b1r, a1r, W2)
    p2 = _agg(hs2, src_r, dst_r, zeros_acc)
    return _k3(p2, hs2, degp, b2r, a2r)
